# baseline (speedup 1.0000x reference)
import functools

import jax
import jax.numpy as jnp
from jax import lax
from jax.experimental import pallas as pl
from jax.experimental.pallas import tpu as pltpu
from jax.experimental.pallas import tpu_sc as plsc

N_NODES = 100000
N_EDGES = 6400000

NC = 2
NS = 16
NW = NC * NS
EPW = N_EDGES // NW
CHUNK = 800
NCHUNK = EPW // CHUNK
LANES = 16
NBUF = 5
LOOK = 2
LAG = NBUF - LOOK
NOUTER = NCHUNK // NBUF
GUNROLL = 5


NPAD = 102400
NSLICE = NPAD // NS


def _make_edge_pass(first):
    mesh = plsc.VectorSubcoreMesh(core_axis_name="c", subcore_axis_name="s")

    out_type = [jax.ShapeDtypeStruct((NC * NPAD,), jnp.float32)]
    if first:
        out_type.append(jax.ShapeDtypeStruct((NC, NPAD), jnp.float32))
    else:
        out_type.append(jax.ShapeDtypeStruct((NPAD,), jnp.float32))

    scratch = [
        pltpu.VMEM((N_NODES,), jnp.float32),
        [pltpu.VMEM((CHUNK,), jnp.int32) for _ in range(NBUF)],
        [pltpu.VMEM((CHUNK,), jnp.int32) for _ in range(NBUF)],
        [pltpu.VMEM((CHUNK,), jnp.float32) for _ in range(NBUF)],
        pltpu.VMEM_SHARED((NPAD,), jnp.float32),
        [pltpu.SemaphoreType.DMA for _ in range(NBUF)],
        [pltpu.SemaphoreType.DMA for _ in range(NBUF)],
    ]
    if first:
        scratch += [
            pltpu.VMEM((CHUNK,), jnp.float32),
            pltpu.VMEM_SHARED((NPAD,), jnp.float32),
            [pltpu.SemaphoreType.DMA for _ in range(NBUF)],
        ]
    else:
        scratch += [
            pltpu.VMEM_SHARED((NPAD,), jnp.float32),
        ]

    @functools.partial(
        pl.kernel,
        mesh=mesh,
        out_type=out_type,
        scratch_types=scratch,
        compiler_params=pltpu.CompilerParams(needs_layout_passes=False),
    )
    def edge_pass(*args):
        if first:
            (e_hbm, row_hbm, col_hbm, sums_hbm, cnt_hbm,
             e_v, row_v, col_v, msg_v, sums_s, in_sem, s_sem,
             ones_v, cnt_s, c_sem) = args
        else:
            (e_hbm, psums_hbm, cnt_hbm, row_hbm, col_hbm, sums_hbm, ecur_hbm,
             e_v, row_v, col_v, msg_v, sums_s, in_sem, s_sem, ecur_s) = args

        cid = lax.axis_index("c")
        sid = lax.axis_index("s")
        wid = sid * NC + cid
        base0 = wid * EPW
        slo = sid * NSLICE

        def zinit(i, c):
            msg_v[0][pl.ds(i * LANES, LANES)] = jnp.zeros((LANES,), jnp.float32)
            return c
        lax.fori_loop(0, CHUNK // LANES, zinit, 0)
        for k in range(NSLICE // CHUNK):
            dst = pl.ds(slo + k * CHUNK, CHUNK)
            pltpu.sync_copy(msg_v[0], sums_s.at[dst])
            if first:
                pltpu.sync_copy(msg_v[0], cnt_s.at[dst])

        if first:
            def ones_body(i, c):
                ones_v[pl.ds(i * LANES, LANES)] = jnp.ones(
                    (LANES,), jnp.float32)
                return c
            lax.fori_loop(0, CHUNK // LANES, ones_body, 0)

            pltpu.sync_copy(e_hbm.at[pl.ds(0, N_NODES)], e_v)
        else:
            R0, R1, R2, R3, R4, R5 = (k * NSLICE for k in range(6))
            sl = pl.ds(slo, NSLICE)
            pltpu.sync_copy(e_hbm.at[sl], e_v.at[pl.ds(R0, NSLICE)])
            pltpu.sync_copy(psums_hbm.at[sl], e_v.at[pl.ds(R1, NSLICE)])
            pltpu.sync_copy(psums_hbm.at[pl.ds(NPAD + slo, NSLICE)],
                            e_v.at[pl.ds(R2, NSLICE)])
            pltpu.sync_copy(cnt_hbm.at[0, sl], e_v.at[pl.ds(R3, NSLICE)])
            pltpu.sync_copy(cnt_hbm.at[1, sl], e_v.at[pl.ds(R4, NSLICE)])

            def upd(g, c):
                off = g * LANES
                ep = e_v[pl.ds(R0 + off, LANES)]
                s01 = (e_v[pl.ds(R1 + off, LANES)]
                       + e_v[pl.ds(R2 + off, LANES)])
                deg = (e_v[pl.ds(R3 + off, LANES)]
                       + e_v[pl.ds(R4 + off, LANES)])
                e_v[pl.ds(R5 + off, LANES)] = (
                    ep * 0.5 + 0.5 * s01 / jnp.maximum(deg, 1.0))
                return c
            lax.fori_loop(0, NSLICE // LANES, upd, 0)

            pltpu.sync_copy(e_v.at[pl.ds(R5, NSLICE)], ecur_s.at[sl])

            @pl.when(cid == 0)
            def _():
                pltpu.sync_copy(e_v.at[pl.ds(R5, NSLICE)], ecur_hbm.at[sl])

            plsc.subcore_barrier()
            pltpu.sync_copy(ecur_s.at[pl.ds(0, N_NODES)], e_v)

        plsc.subcore_barrier()

        def issue_in(ci, b):
            src = pl.ds(base0 + ci * CHUNK, CHUNK)
            pltpu.async_copy(row_hbm.at[src], row_v[b], in_sem[b])
            pltpu.async_copy(col_hbm.at[src], col_v[b], in_sem[b])

        def wait_in(b):
            pltpu.make_async_copy(row_hbm.at[pl.ds(0, CHUNK)], row_v[b],
                                  in_sem[b]).wait()
            pltpu.make_async_copy(col_hbm.at[pl.ds(0, CHUNK)], col_v[b],
                                  in_sem[b]).wait()

        def drain_scatter(b):
            pltpu.make_async_copy(msg_v[b], sums_s.at[col_v[b]],
                                  s_sem[b]).wait()
            if first:
                pltpu.make_async_copy(ones_v, cnt_s.at[col_v[b]],
                                      c_sem[b]).wait()

        for ci in range(LOOK):
            issue_in(ci, ci)

        def outer(j, carry):
            for b in range(NBUF):
                ci = j * NBUF + b
                wait_in(b)

                def gather(g, c):
                    for u in range(GUNROLL):
                        off = (g * GUNROLL + u) * LANES
                        idx = row_v[b][pl.ds(off, LANES)]
                        msg_v[b][pl.ds(off, LANES)] = plsc.load_gather(
                            e_v, [idx])
                    return c
                lax.fori_loop(0, CHUNK // (LANES * GUNROLL), gather, 0)

                pltpu.async_copy(msg_v[b], sums_s.at[col_v[b]], s_sem[b],
                                 add=True)
                if first:
                    pltpu.async_copy(ones_v, cnt_s.at[col_v[b]], c_sem[b],
                                     add=True)

                bn = (b + LOOK) % NBUF
                if b < LAG:
                    @pl.when(ci >= LAG)
                    def _():
                        drain_scatter(bn)
                    issue_in(ci + LOOK, bn)
                else:
                    drain_scatter(bn)

                    @pl.when(ci + LOOK < NCHUNK)
                    def _():
                        issue_in(ci + LOOK, bn)
            return carry

        lax.fori_loop(0, NOUTER, outer, 0)

        for b in range(LOOK, NBUF):
            drain_scatter(b)

        plsc.subcore_barrier()

        sl = pl.ds(slo, NSLICE)
        pltpu.sync_copy(sums_s.at[sl],
                        sums_hbm.at[pl.ds(cid * NPAD + slo, NSLICE)])
        if first:
            pltpu.sync_copy(cnt_s.at[sl], cnt_hbm.at[cid, sl])

    return edge_pass


_edge_pass_first = _make_edge_pass(True)
_edge_pass_upd = _make_edge_pass(False)


def _make_edge_pass_both():
    mesh = plsc.VectorSubcoreMesh(core_axis_name="c", subcore_axis_name="s")

    out_type = [
        jax.ShapeDtypeStruct((NC * NPAD,), jnp.float32),
        jax.ShapeDtypeStruct((NC, NPAD), jnp.float32),
        jax.ShapeDtypeStruct((NPAD,), jnp.float32),
        jax.ShapeDtypeStruct((NC * NPAD,), jnp.float32),
    ]

    scratch = [
        pltpu.VMEM((N_NODES,), jnp.float32),
        [pltpu.VMEM((CHUNK,), jnp.int32) for _ in range(NBUF)],
        [pltpu.VMEM((CHUNK,), jnp.int32) for _ in range(NBUF)],
        [pltpu.VMEM((CHUNK,), jnp.float32) for _ in range(NBUF)],
        pltpu.VMEM_SHARED((NPAD,), jnp.float32),
        [pltpu.SemaphoreType.DMA for _ in range(NBUF)],
        [pltpu.SemaphoreType.DMA for _ in range(NBUF)],
        pltpu.VMEM((CHUNK,), jnp.float32),
        pltpu.VMEM_SHARED((NPAD,), jnp.float32),
        [pltpu.SemaphoreType.DMA for _ in range(NBUF)],
        pltpu.SemaphoreType.REGULAR,
    ]

    @functools.partial(
        pl.kernel,
        mesh=mesh,
        out_type=out_type,
        scratch_types=scratch,
        compiler_params=pltpu.CompilerParams(needs_layout_passes=False),
    )
    def edge_pass_both(e_hbm, row_hbm, col_hbm,
                       s2_hbm, cnt_hbm, e2_hbm, s1_hbm,
                       e_v, row_v, col_v, msg_v, sums_s, in_sem, s_sem,
                       ones_v, cnt_s, c_sem, xsem):
        cid = lax.axis_index("c")
        sid = lax.axis_index("s")
        wid = sid * NC + cid
        base0 = wid * EPW
        slo = sid * NSLICE
        sl = pl.ds(slo, NSLICE)

        def zero_slices(with_cnt):
            def zinit(i, c):
                msg_v[0][pl.ds(i * LANES, LANES)] = jnp.zeros(
                    (LANES,), jnp.float32)
                return c
            lax.fori_loop(0, CHUNK // LANES, zinit, 0)
            for k in range(NSLICE // CHUNK):
                dst = pl.ds(slo + k * CHUNK, CHUNK)
                pltpu.sync_copy(msg_v[0], sums_s.at[dst])
                if with_cnt:
                    pltpu.sync_copy(msg_v[0], cnt_s.at[dst])

        def issue_in(ci, b):
            src = pl.ds(base0 + ci * CHUNK, CHUNK)
            pltpu.async_copy(row_hbm.at[src], row_v[b], in_sem[b])
            pltpu.async_copy(col_hbm.at[src], col_v[b], in_sem[b])

        def wait_in(b):
            pltpu.make_async_copy(row_hbm.at[pl.ds(0, CHUNK)], row_v[b],
                                  in_sem[b]).wait()
            pltpu.make_async_copy(col_hbm.at[pl.ds(0, CHUNK)], col_v[b],
                                  in_sem[b]).wait()

        def edge_loop(with_cnt):
            def drain_scatter(b):
                pltpu.make_async_copy(msg_v[b], sums_s.at[col_v[b]],
                                      s_sem[b]).wait()
                if with_cnt:
                    pltpu.make_async_copy(ones_v, cnt_s.at[col_v[b]],
                                          c_sem[b]).wait()

            for ci in range(LOOK):
                issue_in(ci, ci)

            def outer(j, carry):
                for b in range(NBUF):
                    ci = j * NBUF + b
                    wait_in(b)

                    def gather(g, c):
                        for u in range(GUNROLL):
                            off = (g * GUNROLL + u) * LANES
                            idx = row_v[b][pl.ds(off, LANES)]
                            msg_v[b][pl.ds(off, LANES)] = plsc.load_gather(
                                e_v, [idx])
                        return c
                    lax.fori_loop(0, CHUNK // (LANES * GUNROLL), gather, 0)

                    pltpu.async_copy(msg_v[b], sums_s.at[col_v[b]], s_sem[b],
                                     add=True)
                    if with_cnt:
                        pltpu.async_copy(ones_v, cnt_s.at[col_v[b]], c_sem[b],
                                         add=True)

                    bn = (b + LOOK) % NBUF
                    if b < LAG:
                        @pl.when(ci >= LAG)
                        def _():
                            drain_scatter(bn)
                        issue_in(ci + LOOK, bn)
                    else:
                        drain_scatter(bn)

                        @pl.when(ci + LOOK < NCHUNK)
                        def _():
                            issue_in(ci + LOOK, bn)
                return carry

            lax.fori_loop(0, NOUTER, outer, 0)
            for b in range(LOOK, NBUF):
                drain_scatter(b)

        zero_slices(True)

        def ones_body(i, c):
            ones_v[pl.ds(i * LANES, LANES)] = jnp.ones((LANES,), jnp.float32)
            return c
        lax.fori_loop(0, CHUNK // LANES, ones_body, 0)

        pltpu.sync_copy(e_hbm.at[pl.ds(0, N_NODES)], e_v)
        plsc.subcore_barrier()

        edge_loop(True)
        plsc.subcore_barrier()

        pltpu.sync_copy(sums_s.at[sl],
                        s1_hbm.at[pl.ds(cid * NPAD + slo, NSLICE)])
        pltpu.sync_copy(cnt_s.at[sl], cnt_hbm.at[cid, sl])

        pl.semaphore_signal(xsem, 1, core_index=1 - cid)
        pl.semaphore_wait(xsem, 1)
        plsc.subcore_barrier()

        zero_slices(False)
        R0, R1, R2, R3, R4, R5 = (k * NSLICE for k in range(6))
        pltpu.sync_copy(e_hbm.at[sl], e_v.at[pl.ds(R0, NSLICE)])
        pltpu.sync_copy(s1_hbm.at[sl], e_v.at[pl.ds(R1, NSLICE)])
        pltpu.sync_copy(s1_hbm.at[pl.ds(NPAD + slo, NSLICE)],
                        e_v.at[pl.ds(R2, NSLICE)])
        pltpu.sync_copy(cnt_hbm.at[0, sl], e_v.at[pl.ds(R3, NSLICE)])
        pltpu.sync_copy(cnt_hbm.at[1, sl], e_v.at[pl.ds(R4, NSLICE)])

        def upd(g, c):
            off = g * LANES
            ep = e_v[pl.ds(R0 + off, LANES)]
            s01 = e_v[pl.ds(R1 + off, LANES)] + e_v[pl.ds(R2 + off, LANES)]
            deg = e_v[pl.ds(R3 + off, LANES)] + e_v[pl.ds(R4 + off, LANES)]
            e_v[pl.ds(R5 + off, LANES)] = (
                ep * 0.5 + 0.5 * s01 / jnp.maximum(deg, 1.0))
            return c
        lax.fori_loop(0, NSLICE // LANES, upd, 0)

        pltpu.sync_copy(e_v.at[pl.ds(R5, NSLICE)], cnt_s.at[sl])

        plsc.subcore_barrier()
        pltpu.sync_copy(cnt_s.at[pl.ds(0, N_NODES)], e_v)

        edge_loop(False)
        plsc.subcore_barrier()

        pltpu.sync_copy(sums_s.at[sl],
                        s2_hbm.at[pl.ds(cid * NPAD + slo, NSLICE)])

        pl.semaphore_signal(xsem, 1, core_index=1 - cid)
        pl.semaphore_wait(xsem, 1)
        plsc.subcore_barrier()

        def save_slice(g, c):
            off = g * LANES
            e_v[pl.ds(R0 + off, LANES)] = e_v[pl.ds(slo + off, LANES)]
            return c
        lax.fori_loop(0, NSLICE // LANES, save_slice, 0)
        pltpu.sync_copy(s2_hbm.at[sl], e_v.at[pl.ds(R1, NSLICE)])
        pltpu.sync_copy(s2_hbm.at[pl.ds(NPAD + slo, NSLICE)],
                        e_v.at[pl.ds(R2, NSLICE)])
        pltpu.sync_copy(cnt_hbm.at[0, sl], e_v.at[pl.ds(R3, NSLICE)])
        pltpu.sync_copy(cnt_hbm.at[1, sl], e_v.at[pl.ds(R4, NSLICE)])
        lax.fori_loop(0, NSLICE // LANES, upd, 0)

        @pl.when(cid == 0)
        def _():
            pltpu.sync_copy(e_v.at[pl.ds(R5, NSLICE)], e2_hbm.at[sl])

    return edge_pass_both


_edge_pass_both = _make_edge_pass_both()

_R = NPAD // 128


def _update_body(e_ref, s_ref, c_ref, o_ref):
    agg = s_ref[0] + s_ref[1]
    deg = c_ref[0] + c_ref[1]
    o_ref[...] = e_ref[...] * 0.5 + 0.5 * agg / jnp.maximum(deg, 1.0)


def _update(e_pad, sums, cnt):
    out = pl.pallas_call(
        _update_body,
        out_shape=jax.ShapeDtypeStruct((_R, 128), jnp.float32),
    )(e_pad.reshape(_R, 128), sums.reshape(NC, _R, 128),
      cnt.reshape(NC, _R, 128))
    return out.reshape(NPAD)


def _as_tuple(r):
    return tuple(r) if isinstance(r, (list, tuple)) else (r,)


def kernel(e, edge_index, l):
    row = edge_index[0]
    col = edge_index[1]
    e_pad = jnp.pad(e, (0, NPAD - N_NODES))

    def two_passes(_):
        _s2, _cnt, e2, _s1 = _as_tuple(_edge_pass_both(e_pad, row, col))
        return e2

    def general(_):
        def body(i, carry):
            e_prev, sums, cnt = carry

            def first(_):
                s, c = _as_tuple(_edge_pass_first(e_prev, row, col))
                return e_prev, s, c

            def later(_):
                s, e_cur = _as_tuple(
                    _edge_pass_upd(e_prev, sums, cnt, row, col))
                return e_cur, s, cnt

            return lax.cond(i == 0, first, later, None)

        sums0 = jnp.zeros((NC * NPAD,), jnp.float32)
        cnt0 = jnp.zeros((NC, NPAD), jnp.float32)
        e_last, sums_l, cnt = lax.fori_loop(0, l, body, (e_pad, sums0, cnt0))

        return lax.cond(
            l >= 1,
            lambda: _update(e_last, sums_l, cnt),
            lambda: e_pad,
        )

    e_out = lax.cond(jnp.equal(l, 2), two_passes, general, None)
    return e_out[:N_NODES]


# speedup vs baseline: 1.0107x; 1.0107x over previous
import functools

import jax
import jax.numpy as jnp
from jax import lax
from jax.experimental import pallas as pl
from jax.experimental.pallas import tpu as pltpu
from jax.experimental.pallas import tpu_sc as plsc

N_NODES = 100000
N_EDGES = 6400000

NC = 2
NS = 16
NW = NC * NS
EPW = N_EDGES // NW
CHUNK = 800
NCHUNK = EPW // CHUNK
LANES = 16
NBUF = 5
LOOK = 2
LAG = NBUF - LOOK
NOUTER = NCHUNK // NBUF
GUNROLL = 5


NPAD = 102400
NSLICE = NPAD // NS


def _make_edge_pass(first):
    mesh = plsc.VectorSubcoreMesh(core_axis_name="c", subcore_axis_name="s")

    out_type = [jax.ShapeDtypeStruct((NC * NPAD,), jnp.float32)]
    if first:
        out_type.append(jax.ShapeDtypeStruct((NC, NPAD), jnp.float32))
    else:
        out_type.append(jax.ShapeDtypeStruct((NPAD,), jnp.float32))

    scratch = [
        pltpu.VMEM((N_NODES,), jnp.float32),
        [pltpu.VMEM((CHUNK,), jnp.int32) for _ in range(NBUF)],
        [pltpu.VMEM((CHUNK,), jnp.int32) for _ in range(NBUF)],
        [pltpu.VMEM((CHUNK,), jnp.float32) for _ in range(NBUF)],
        pltpu.VMEM_SHARED((NPAD,), jnp.float32),
        [pltpu.SemaphoreType.DMA for _ in range(NBUF)],
        [pltpu.SemaphoreType.DMA for _ in range(NBUF)],
    ]
    if first:
        scratch += [
            pltpu.VMEM((CHUNK,), jnp.float32),
            pltpu.VMEM_SHARED((NPAD,), jnp.float32),
            [pltpu.SemaphoreType.DMA for _ in range(NBUF)],
        ]
    else:
        scratch += [
            pltpu.VMEM_SHARED((NPAD,), jnp.float32),
        ]

    @functools.partial(
        pl.kernel,
        mesh=mesh,
        out_type=out_type,
        scratch_types=scratch,
        compiler_params=pltpu.CompilerParams(needs_layout_passes=False),
    )
    def edge_pass(*args):
        if first:
            (e_hbm, row_hbm, col_hbm, sums_hbm, cnt_hbm,
             e_v, row_v, col_v, msg_v, sums_s, in_sem, s_sem,
             ones_v, cnt_s, c_sem) = args
        else:
            (e_hbm, psums_hbm, cnt_hbm, row_hbm, col_hbm, sums_hbm, ecur_hbm,
             e_v, row_v, col_v, msg_v, sums_s, in_sem, s_sem, ecur_s) = args

        cid = lax.axis_index("c")
        sid = lax.axis_index("s")
        wid = sid * NC + cid
        base0 = wid * EPW
        slo = sid * NSLICE

        def zinit(i, c):
            msg_v[0][pl.ds(i * LANES, LANES)] = jnp.zeros((LANES,), jnp.float32)
            return c
        lax.fori_loop(0, CHUNK // LANES, zinit, 0)
        for k in range(NSLICE // CHUNK):
            dst = pl.ds(slo + k * CHUNK, CHUNK)
            pltpu.sync_copy(msg_v[0], sums_s.at[dst])
            if first:
                pltpu.sync_copy(msg_v[0], cnt_s.at[dst])

        if first:
            def ones_body(i, c):
                ones_v[pl.ds(i * LANES, LANES)] = jnp.ones(
                    (LANES,), jnp.float32)
                return c
            lax.fori_loop(0, CHUNK // LANES, ones_body, 0)

            pltpu.sync_copy(e_hbm.at[pl.ds(0, N_NODES)], e_v)
        else:
            R0, R1, R2, R3, R4, R5 = (k * NSLICE for k in range(6))
            sl = pl.ds(slo, NSLICE)
            pltpu.sync_copy(e_hbm.at[sl], e_v.at[pl.ds(R0, NSLICE)])
            pltpu.sync_copy(psums_hbm.at[sl], e_v.at[pl.ds(R1, NSLICE)])
            pltpu.sync_copy(psums_hbm.at[pl.ds(NPAD + slo, NSLICE)],
                            e_v.at[pl.ds(R2, NSLICE)])
            pltpu.sync_copy(cnt_hbm.at[0, sl], e_v.at[pl.ds(R3, NSLICE)])
            pltpu.sync_copy(cnt_hbm.at[1, sl], e_v.at[pl.ds(R4, NSLICE)])

            def upd(g, c):
                off = g * LANES
                ep = e_v[pl.ds(R0 + off, LANES)]
                s01 = (e_v[pl.ds(R1 + off, LANES)]
                       + e_v[pl.ds(R2 + off, LANES)])
                deg = (e_v[pl.ds(R3 + off, LANES)]
                       + e_v[pl.ds(R4 + off, LANES)])
                e_v[pl.ds(R5 + off, LANES)] = (
                    ep * 0.5 + 0.5 * s01 / jnp.maximum(deg, 1.0))
                return c
            lax.fori_loop(0, NSLICE // LANES, upd, 0)

            pltpu.sync_copy(e_v.at[pl.ds(R5, NSLICE)], ecur_s.at[sl])

            @pl.when(cid == 0)
            def _():
                pltpu.sync_copy(e_v.at[pl.ds(R5, NSLICE)], ecur_hbm.at[sl])

            plsc.subcore_barrier()
            pltpu.sync_copy(ecur_s.at[pl.ds(0, N_NODES)], e_v)

        plsc.subcore_barrier()

        def issue_in(ci, b):
            src = pl.ds(base0 + ci * CHUNK, CHUNK)
            pltpu.async_copy(row_hbm.at[src], row_v[b], in_sem[b])
            pltpu.async_copy(col_hbm.at[src], col_v[b], in_sem[b])

        def wait_in(b):
            pltpu.make_async_copy(row_hbm.at[pl.ds(0, CHUNK)], row_v[b],
                                  in_sem[b]).wait()
            pltpu.make_async_copy(col_hbm.at[pl.ds(0, CHUNK)], col_v[b],
                                  in_sem[b]).wait()

        def drain_scatter(b):
            pltpu.make_async_copy(msg_v[b], sums_s.at[col_v[b]],
                                  s_sem[b]).wait()
            if first:
                pltpu.make_async_copy(ones_v, cnt_s.at[col_v[b]],
                                      c_sem[b]).wait()

        for ci in range(LOOK):
            issue_in(ci, ci)

        def outer(j, carry):
            for b in range(NBUF):
                ci = j * NBUF + b
                wait_in(b)

                def gather(g, c):
                    for u in range(GUNROLL):
                        off = (g * GUNROLL + u) * LANES
                        idx = row_v[b][pl.ds(off, LANES)]
                        msg_v[b][pl.ds(off, LANES)] = plsc.load_gather(
                            e_v, [idx])
                    return c
                lax.fori_loop(0, CHUNK // (LANES * GUNROLL), gather, 0)

                pltpu.async_copy(msg_v[b], sums_s.at[col_v[b]], s_sem[b],
                                 add=True)
                if first:
                    pltpu.async_copy(ones_v, cnt_s.at[col_v[b]], c_sem[b],
                                     add=True)

                bn = (b + LOOK) % NBUF
                if b < LAG:
                    @pl.when(ci >= LAG)
                    def _():
                        drain_scatter(bn)
                    issue_in(ci + LOOK, bn)
                else:
                    drain_scatter(bn)

                    @pl.when(ci + LOOK < NCHUNK)
                    def _():
                        issue_in(ci + LOOK, bn)
            return carry

        lax.fori_loop(0, NOUTER, outer, 0)

        for b in range(LOOK, NBUF):
            drain_scatter(b)

        plsc.subcore_barrier()

        sl = pl.ds(slo, NSLICE)
        pltpu.sync_copy(sums_s.at[sl],
                        sums_hbm.at[pl.ds(cid * NPAD + slo, NSLICE)])
        if first:
            pltpu.sync_copy(cnt_s.at[sl], cnt_hbm.at[cid, sl])

    return edge_pass


_edge_pass_first = _make_edge_pass(True)
_edge_pass_upd = _make_edge_pass(False)


def _make_edge_pass_both():
    mesh = plsc.VectorSubcoreMesh(core_axis_name="c", subcore_axis_name="s")

    out_type = [
        jax.ShapeDtypeStruct((NC * NPAD,), jnp.float32),
        jax.ShapeDtypeStruct((NC, NPAD), jnp.float32),
        jax.ShapeDtypeStruct((NPAD,), jnp.float32),
        jax.ShapeDtypeStruct((NC * NPAD,), jnp.float32),
    ]

    scratch = [
        pltpu.VMEM((N_NODES,), jnp.float32),
        [pltpu.VMEM((CHUNK,), jnp.int32) for _ in range(NBUF)],
        [pltpu.VMEM((CHUNK,), jnp.int32) for _ in range(NBUF)],
        [pltpu.VMEM((CHUNK,), jnp.float32) for _ in range(NBUF)],
        pltpu.VMEM_SHARED((NPAD,), jnp.float32),
        [pltpu.SemaphoreType.DMA for _ in range(NBUF)],
        [pltpu.SemaphoreType.DMA for _ in range(NBUF)],
        pltpu.VMEM((CHUNK,), jnp.float32),
        pltpu.VMEM_SHARED((NPAD,), jnp.float32),
        [pltpu.SemaphoreType.DMA for _ in range(NBUF)],
        pltpu.SemaphoreType.REGULAR,
    ]

    @functools.partial(
        pl.kernel,
        mesh=mesh,
        out_type=out_type,
        scratch_types=scratch,
        compiler_params=pltpu.CompilerParams(needs_layout_passes=False),
    )
    def edge_pass_both(e_hbm, row_hbm, col_hbm,
                       s2_hbm, cnt_hbm, e1_hbm, s1_hbm,
                       e_v, row_v, col_v, msg_v, sums_s, in_sem, s_sem,
                       ones_v, cnt_s, c_sem, xsem):
        cid = lax.axis_index("c")
        sid = lax.axis_index("s")
        wid = sid * NC + cid
        base0 = wid * EPW
        slo = sid * NSLICE
        sl = pl.ds(slo, NSLICE)

        def zero_slices(with_cnt):
            def zinit(i, c):
                msg_v[0][pl.ds(i * LANES, LANES)] = jnp.zeros(
                    (LANES,), jnp.float32)
                return c
            lax.fori_loop(0, CHUNK // LANES, zinit, 0)
            for k in range(NSLICE // CHUNK):
                dst = pl.ds(slo + k * CHUNK, CHUNK)
                pltpu.sync_copy(msg_v[0], sums_s.at[dst])
                if with_cnt:
                    pltpu.sync_copy(msg_v[0], cnt_s.at[dst])

        def issue_in(ci, b):
            src = pl.ds(base0 + ci * CHUNK, CHUNK)
            pltpu.async_copy(row_hbm.at[src], row_v[b], in_sem[b])
            pltpu.async_copy(col_hbm.at[src], col_v[b], in_sem[b])

        def wait_in(b):
            pltpu.make_async_copy(row_hbm.at[pl.ds(0, CHUNK)], row_v[b],
                                  in_sem[b]).wait()
            pltpu.make_async_copy(col_hbm.at[pl.ds(0, CHUNK)], col_v[b],
                                  in_sem[b]).wait()

        def edge_loop(with_cnt):
            def drain_scatter(b):
                pltpu.make_async_copy(msg_v[b], sums_s.at[col_v[b]],
                                      s_sem[b]).wait()
                if with_cnt:
                    pltpu.make_async_copy(ones_v, cnt_s.at[col_v[b]],
                                          c_sem[b]).wait()

            for ci in range(LOOK):
                issue_in(ci, ci)

            def outer(j, carry):
                for b in range(NBUF):
                    ci = j * NBUF + b
                    wait_in(b)

                    def gather(g, c):
                        for u in range(GUNROLL):
                            off = (g * GUNROLL + u) * LANES
                            idx = row_v[b][pl.ds(off, LANES)]
                            msg_v[b][pl.ds(off, LANES)] = plsc.load_gather(
                                e_v, [idx])
                        return c
                    lax.fori_loop(0, CHUNK // (LANES * GUNROLL), gather, 0)

                    pltpu.async_copy(msg_v[b], sums_s.at[col_v[b]], s_sem[b],
                                     add=True)
                    if with_cnt:
                        pltpu.async_copy(ones_v, cnt_s.at[col_v[b]], c_sem[b],
                                         add=True)

                    bn = (b + LOOK) % NBUF
                    if b < LAG:
                        @pl.when(ci >= LAG)
                        def _():
                            drain_scatter(bn)
                        issue_in(ci + LOOK, bn)
                    else:
                        drain_scatter(bn)

                        @pl.when(ci + LOOK < NCHUNK)
                        def _():
                            issue_in(ci + LOOK, bn)
                return carry

            lax.fori_loop(0, NOUTER, outer, 0)
            for b in range(LOOK, NBUF):
                drain_scatter(b)

        zero_slices(True)

        def ones_body(i, c):
            ones_v[pl.ds(i * LANES, LANES)] = jnp.ones((LANES,), jnp.float32)
            return c
        lax.fori_loop(0, CHUNK // LANES, ones_body, 0)

        pltpu.sync_copy(e_hbm.at[pl.ds(0, N_NODES)], e_v)
        plsc.subcore_barrier()

        edge_loop(True)
        plsc.subcore_barrier()

        pltpu.sync_copy(sums_s.at[sl],
                        s1_hbm.at[pl.ds(cid * NPAD + slo, NSLICE)])
        pltpu.sync_copy(cnt_s.at[sl], cnt_hbm.at[cid, sl])

        pl.semaphore_signal(xsem, 1, core_index=1 - cid)
        pl.semaphore_wait(xsem, 1)
        plsc.subcore_barrier()

        zero_slices(False)
        R0, R1, R2, R3, R4, R5 = (k * NSLICE for k in range(6))
        pltpu.sync_copy(e_hbm.at[sl], e_v.at[pl.ds(R0, NSLICE)])
        pltpu.sync_copy(s1_hbm.at[sl], e_v.at[pl.ds(R1, NSLICE)])
        pltpu.sync_copy(s1_hbm.at[pl.ds(NPAD + slo, NSLICE)],
                        e_v.at[pl.ds(R2, NSLICE)])
        pltpu.sync_copy(cnt_hbm.at[0, sl], e_v.at[pl.ds(R3, NSLICE)])
        pltpu.sync_copy(cnt_hbm.at[1, sl], e_v.at[pl.ds(R4, NSLICE)])

        def upd(g, c):
            off = g * LANES
            ep = e_v[pl.ds(R0 + off, LANES)]
            s01 = e_v[pl.ds(R1 + off, LANES)] + e_v[pl.ds(R2 + off, LANES)]
            deg = e_v[pl.ds(R3 + off, LANES)] + e_v[pl.ds(R4 + off, LANES)]
            e_v[pl.ds(R5 + off, LANES)] = (
                ep * 0.5 + 0.5 * s01 / jnp.maximum(deg, 1.0))
            return c
        lax.fori_loop(0, NSLICE // LANES, upd, 0)

        pltpu.sync_copy(e_v.at[pl.ds(R5, NSLICE)], cnt_s.at[sl])

        @pl.when(cid == 0)
        def _():
            pltpu.sync_copy(e_v.at[pl.ds(R5, NSLICE)], e1_hbm.at[sl])

        plsc.subcore_barrier()
        pltpu.sync_copy(cnt_s.at[pl.ds(0, N_NODES)], e_v)

        edge_loop(False)
        plsc.subcore_barrier()

        pltpu.sync_copy(sums_s.at[sl],
                        s2_hbm.at[pl.ds(cid * NPAD + slo, NSLICE)])

    return edge_pass_both


_edge_pass_both = _make_edge_pass_both()

_R = NPAD // 128


def _update_body(e_ref, s_ref, c_ref, o_ref):
    agg = s_ref[0] + s_ref[1]
    deg = c_ref[0] + c_ref[1]
    o_ref[...] = e_ref[...] * 0.5 + 0.5 * agg / jnp.maximum(deg, 1.0)


def _update(e_pad, sums, cnt):
    out = pl.pallas_call(
        _update_body,
        out_shape=jax.ShapeDtypeStruct((_R, 128), jnp.float32),
    )(e_pad.reshape(_R, 128), sums.reshape(NC, _R, 128),
      cnt.reshape(NC, _R, 128))
    return out.reshape(NPAD)


def _as_tuple(r):
    return tuple(r) if isinstance(r, (list, tuple)) else (r,)


def kernel(e, edge_index, l):
    row = edge_index[0]
    col = edge_index[1]
    e_pad = jnp.pad(e, (0, NPAD - N_NODES))

    def two_passes(_):
        s2, cnt, e1, _s1 = _as_tuple(_edge_pass_both(e_pad, row, col))
        return _update(e1, s2, cnt)

    def general(_):
        def body(i, carry):
            e_prev, sums, cnt = carry

            def first(_):
                s, c = _as_tuple(_edge_pass_first(e_prev, row, col))
                return e_prev, s, c

            def later(_):
                s, e_cur = _as_tuple(
                    _edge_pass_upd(e_prev, sums, cnt, row, col))
                return e_cur, s, cnt

            return lax.cond(i == 0, first, later, None)

        sums0 = jnp.zeros((NC * NPAD,), jnp.float32)
        cnt0 = jnp.zeros((NC, NPAD), jnp.float32)
        e_last, sums_l, cnt = lax.fori_loop(0, l, body, (e_pad, sums0, cnt0))

        return lax.cond(
            l >= 1,
            lambda: _update(e_last, sums_l, cnt),
            lambda: e_pad,
        )

    e_out = lax.cond(jnp.equal(l, 2), two_passes, general, None)
    return e_out[:N_NODES]
